# Initial kernel scaffold; baseline (speedup 1.0000x reference)
import jax
import jax.numpy as jnp
from jax.experimental import pallas as pl


def kernel(x1, edge_index_1, batch1, x2, edge_index_2, batch2, target, params):
    raise NotImplementedError("write your pallas kernel here")


# speedup vs baseline: 3.4580x; 3.4580x over previous
import functools

import jax
import jax.numpy as jnp
from jax import lax
from jax.experimental import pallas as pl
from jax.experimental.pallas import tpu as pltpu
from jax.experimental.pallas import tpu_sc as plsc

_N = 50000
_B = 512
_DIM = 32
_FEAT = 78
_EPS = 1e-5

_NC = 2
_NS = 16
_NW = _NC * _NS
_CHUNK = 128


@functools.lru_cache(maxsize=None)
def _make_sc_scatter(e_len, n_acc, nbuf=2):
    rpt = n_acc // _NW
    acc_t = n_acc // _NS
    zrows = min(128, acc_t)
    dummy = n_acc - 1

    mesh = plsc.VectorSubcoreMesh(
        core_axis_name="c", subcore_axis_name="s",
        num_cores=_NC, num_subcores=_NS)

    scratch = [pltpu.VMEM_SHARED((n_acc, _DIM), jnp.float32),
               pltpu.VMEM((zrows, _DIM), jnp.float32),
               pltpu.VMEM((48,), jnp.int32)]
    scratch += [pltpu.VMEM((_CHUNK,), jnp.int32) for _ in range(nbuf)]
    scratch += [pltpu.VMEM((_CHUNK,), jnp.int32) for _ in range(nbuf)]
    scratch += [pltpu.VMEM((_CHUNK, _DIM), jnp.float32) for _ in range(nbuf)]
    scratch += [pltpu.SemaphoreType.DMA for _ in range(2 * nbuf)]

    out_sds = jax.ShapeDtypeStruct((n_acc, _DIM), jnp.float32)

    @functools.partial(
        pl.kernel,
        out_type=out_sds,
        mesh=mesh,
        scratch_types=tuple(scratch),
        compiler_params=pltpu.CompilerParams(use_tc_tiling_on_sc=False),
    )
    def kern(src_hbm, dst_hbm, eb_hbm, table_hbm, out_hbm,
             acc, zbuf, ebs, *bufs):
        si = bufs[0:nbuf]
        di = bufs[nbuf:2 * nbuf]
        rb = bufs[2 * nbuf:3 * nbuf]
        gsem = bufs[3 * nbuf:4 * nbuf]
        ssem = bufs[4 * nbuf:5 * nbuf]

        c = lax.axis_index("c")
        s = lax.axis_index("s")
        wid = c * _NS + s

        pltpu.sync_copy(eb_hbm, ebs)

        def zfill(r, _):
            zbuf[r, pl.ds(0, 16)] = jnp.zeros((16,), jnp.float32)
            zbuf[r, pl.ds(16, 16)] = jnp.zeros((16,), jnp.float32)
            return 0
        lax.fori_loop(0, zrows, zfill, 0)
        z0 = s * acc_t
        done = 0
        while done < acc_t:
            step = min(zrows, acc_t - done)
            pltpu.sync_copy(zbuf.at[pl.ds(0, step)],
                            acc.at[pl.ds(z0 + done, step)])
            done += step
        plsc.subcore_barrier()

        ebv = ebs[pl.ds(wid, 16)]
        lo = ebv[0]
        hi = ebv[1]
        a0 = (lo // 8) * 8
        ngroups = ((hi - a0 + _CHUNK - 1) // _CHUNK + (nbuf - 1)) // nbuf

        def load_masked(b, off):
            offm = pl.multiple_of(off, 8)
            pltpu.sync_copy(src_hbm.at[pl.ds(offm, _CHUNK)], si[b])
            pltpu.sync_copy(dst_hbm.at[pl.ds(offm, _CHUNK)], di[b])
            for i in range(_CHUNK // 16):
                pos = lax.iota(jnp.int32, 16) + (off + 16 * i)
                ok = (pos >= lo) & (pos < hi)
                di[b][pl.ds(16 * i, 16)] = jnp.where(
                    ok, di[b][pl.ds(16 * i, 16)], dummy)
                si[b][pl.ds(16 * i, 16)] = jnp.where(
                    ok, si[b][pl.ds(16 * i, 16)], 0)

        def group(g, _):
            gd = []
            for b in range(nbuf):
                off = a0 + (g * nbuf + b) * _CHUNK
                load_masked(b, off)
                gd.append(pltpu.async_copy(
                    table_hbm.at[si[b]], rb[b], gsem[b]))
            sd = []
            for b in range(nbuf):
                gd[b].wait()
                sd.append(pltpu.async_copy(rb[b], acc.at[di[b]], ssem[b],
                                           add=True))
            for b in range(nbuf):
                sd[b].wait()
            return 0

        lax.fori_loop(0, ngroups, group, 0)
        plsc.subcore_barrier()

        start = pl.multiple_of(wid * rpt, 8)
        pltpu.sync_copy(acc.at[pl.ds(start, rpt)],
                        out_hbm.at[pl.ds(start, rpt)])

    return kern


_E_LEN = 800384
_ACC_N = 51200
_POOL_E_LEN = 50384
_POOL_ACC = 1024


def _edge_scatter(src_s, dst_s, eb, table):
    return _make_sc_scatter(_E_LEN, _ACC_N)(src_s, dst_s, eb, table)


def _pool_scatter(src_s, dst_s, eb, table):
    return _make_sc_scatter(_POOL_E_LEN, _POOL_ACC)(src_s, dst_s, eb, table)


def _csr_prep(src, dst, e_len, rpt):
    order = jnp.argsort(dst, stable=True)
    src_s = src[order]
    dst_s = dst[order]
    eb = jnp.searchsorted(
        dst_s, jnp.arange(33, dtype=jnp.int32) * rpt).astype(jnp.int32)
    eb = jnp.concatenate([eb, jnp.zeros((15,), jnp.int32)])
    pad = e_len - src.shape[0]
    src_s = jnp.concatenate([src_s, jnp.zeros((pad,), jnp.int32)])
    dst_s = jnp.concatenate([dst_s, jnp.zeros((pad,), jnp.int32)])
    return src_s, dst_s, eb


_RB = 5000


def _conv(x, aggs, feat, w1, b1, w2, b2):
    n_slabs = len(aggs)

    def f(*refs):
        x_ref = refs[0]
        a_refs = refs[1:1 + n_slabs]
        b1_ref, w1_ref, b2_ref, w2_ref = refs[1 + n_slabs:5 + n_slabs]
        u_ref, s1_ref, s2_ref = refs[5 + n_slabs:]
        slabs = [a[...] for a in a_refs]
        agg = slabs[0] if n_slabs == 1 else jnp.concatenate(slabs, axis=1)
        t = x_ref[...] + agg[:, :feat]
        y1 = jnp.maximum(jnp.dot(t, w1_ref[...],
                                 preferred_element_type=jnp.float32)
                         + b1_ref[...], 0.0)
        h = jnp.dot(y1, w2_ref[...], preferred_element_type=jnp.float32) \
            + b2_ref[...]
        u = jnp.maximum(h, 0.0)
        u_ref[...] = u

        @pl.when(pl.program_id(0) == 0)
        def _():
            s1_ref[...] = jnp.zeros_like(s1_ref)
            s2_ref[...] = jnp.zeros_like(s2_ref)
        s1_ref[...] += jnp.sum(u, axis=0, keepdims=True)
        s2_ref[...] += jnp.sum(u * u, axis=0, keepdims=True)

    node_in = pl.BlockSpec((_RB, feat), lambda i: (i, 0))
    node = pl.BlockSpec((_RB, _DIM), lambda i: (i, 0))
    vec = pl.BlockSpec((1, _DIM), lambda i: (0, 0))
    mat1 = pl.BlockSpec((feat, _DIM), lambda i: (0, 0))
    mat2 = pl.BlockSpec((_DIM, _DIM), lambda i: (0, 0))
    return pl.pallas_call(
        f,
        grid=(_N // _RB,),
        in_specs=[node_in] + [node] * n_slabs + [vec, mat1, vec, mat2],
        out_specs=(node, vec, vec),
        out_shape=(jax.ShapeDtypeStruct((_N, _DIM), jnp.float32),
                   jax.ShapeDtypeStruct((1, _DIM), jnp.float32),
                   jax.ShapeDtypeStruct((1, _DIM), jnp.float32)),
    )(x, *aggs, b1.reshape(1, _DIM), w1, b2.reshape(1, _DIM), w2)


def _bn_affine(u, s1, s2, gamma, beta):
    def f(u_ref, s1_ref, s2_ref, g_ref, be_ref, y_ref):
        mu = s1_ref[...] * (1.0 / _N)
        var = s2_ref[...] * (1.0 / _N) - mu * mu
        y_ref[...] = g_ref[...] * (u_ref[...] - mu) \
            / jnp.sqrt(var + _EPS) + be_ref[...]

    node = pl.BlockSpec((_RB, _DIM), lambda i: (i, 0))
    vec = pl.BlockSpec((1, _DIM), lambda i: (0, 0))
    return pl.pallas_call(
        f,
        grid=(_N // _RB,),
        in_specs=[node, vec, vec, vec, vec],
        out_specs=node,
        out_shape=jax.ShapeDtypeStruct((_N, _DIM), jnp.float32),
    )(u, s1, s2, gamma.reshape(1, _DIM), beta.reshape(1, _DIM))


def _head(p1, p2, target, params):
    d1, d2 = params["D1"], params["D2"]
    ins = [p1, p2, target,
           d1["fc"]["W"], d1["fc"]["b"].reshape(1, -1),
           d2["fc"]["W"], d2["fc"]["b"].reshape(1, -1),
           params["fc1_xt"]["W"], params["fc1_xt"]["b"].reshape(1, -1),
           params["fc1"]["W"], params["fc1"]["b"].reshape(1, -1),
           params["fc2"]["W"], params["fc2"]["b"].reshape(1, -1),
           params["out"]["W"], params["out"]["b"].reshape(1, -1)]

    def f(p1_ref, p2_ref, tg_ref,
          wf1, bf1, wf2, bf2, wxt, bxt, w1, b1_, w2, b2_, wo, bo, out_ref):
        dot = functools.partial(jnp.dot, preferred_element_type=jnp.float32)
        h1 = jnp.maximum(dot(p1_ref[...], wf1[...]) + bf1[...], 0.0)
        h2 = jnp.maximum(dot(p2_ref[...], wf2[...]) + bf2[...], 0.0)
        xt = dot(tg_ref[...], wxt[...]) + bxt[...]
        xc = jnp.concatenate([h1, h2, xt], axis=1)
        z = jnp.maximum(dot(xc, w1[...]) + b1_[...], 0.0)
        z = jnp.maximum(dot(z, w2[...]) + b2_[...], 0.0)
        out_ref[...] = dot(z, wo[...]) + bo[...]

    return pl.pallas_call(
        f,
        out_shape=jax.ShapeDtypeStruct((_B, 1), jnp.float32),
    )(*ins)


def _branch(x, edge_index, batch, bp):
    src_s, dst_s, eb = _csr_prep(edge_index[0], edge_index[1],
                                 _E_LEN, _ACC_N // _NW)
    layers = bp["layers"]

    xp = jnp.pad(x, ((0, 0), (0, 96 - _FEAT)))
    aggs = [_edge_scatter(src_s, dst_s, eb, xp[:, 32 * k:32 * (k + 1)])
            for k in range(3)]
    lp = layers[0]
    u, s1, s2 = _conv(x, aggs, _FEAT, lp["lin1"]["W"], lp["lin1"]["b"],
                      lp["lin2"]["W"], lp["lin2"]["b"])
    for li in range(1, 5):
        lp = layers[li]
        prev = layers[li - 1]
        xl = _bn_affine(u, s1, s2, prev["gamma"], prev["beta"])
        agg = _edge_scatter(src_s, dst_s, eb, xl)
        u, s1, s2 = _conv(xl, [agg], _DIM, lp["lin1"]["W"], lp["lin1"]["b"],
                          lp["lin2"]["W"], lp["lin2"]["b"])
    x5 = _bn_affine(u, s1, s2, layers[4]["gamma"], layers[4]["beta"])

    psrc = jnp.arange(_N, dtype=jnp.int32)
    peb = jnp.searchsorted(
        batch, jnp.arange(33, dtype=jnp.int32) * (_POOL_ACC // _NW)
    ).astype(jnp.int32)
    peb = jnp.concatenate([peb, jnp.zeros((15,), jnp.int32)])
    pad = _POOL_E_LEN - _N
    psrc = jnp.concatenate([psrc, jnp.zeros((pad,), jnp.int32)])
    pdst = jnp.concatenate([batch, jnp.zeros((pad,), jnp.int32)])
    pooled = _pool_scatter(psrc, pdst, peb, x5)
    return pooled[:_B]


def kernel(x1, edge_index_1, batch1, x2, edge_index_2, batch2, target, params):
    p1 = _branch(x1, edge_index_1, batch1, params["D1"])
    p2 = _branch(x2, edge_index_2, batch2, params["D2"])
    return _head(p1, p2, target, params)


# speedup vs baseline: 4.9084x; 1.4194x over previous
import functools

import jax
import jax.numpy as jnp
from jax import lax
from jax.experimental import pallas as pl
from jax.experimental.pallas import tpu as pltpu
from jax.experimental.pallas import tpu_sc as plsc

_N = 50000
_B = 512
_DIM = 32
_FEAT = 78
_EPS = 1e-5

_NC = 2
_NS = 16
_NW = _NC * _NS
_CHUNK = 128


@functools.lru_cache(maxsize=None)
def _make_sc_scatter(e_len, n_acc, nbuf=4):
    rpt = n_acc // _NW
    acc_t = n_acc // _NS
    zrows = min(128, acc_t)
    dummy = n_acc - 1

    mesh = plsc.VectorSubcoreMesh(
        core_axis_name="c", subcore_axis_name="s",
        num_cores=_NC, num_subcores=_NS)

    scratch = [pltpu.VMEM_SHARED((n_acc, _DIM), jnp.float32),
               pltpu.VMEM((zrows, _DIM), jnp.float32),
               pltpu.VMEM((48,), jnp.int32),
               pltpu.VMEM((8 * _CHUNK,), jnp.int32),
               pltpu.VMEM((8 * _CHUNK,), jnp.int32)]
    scratch += [pltpu.VMEM((_CHUNK,), jnp.int32) for _ in range(nbuf)]
    scratch += [pltpu.VMEM((_CHUNK,), jnp.int32) for _ in range(nbuf)]
    scratch += [pltpu.VMEM((_CHUNK, _DIM), jnp.float32) for _ in range(nbuf)]
    scratch += [pltpu.SemaphoreType.DMA for _ in range(2 * nbuf)]

    out_sds = jax.ShapeDtypeStruct((n_acc, _DIM), jnp.float32)

    @functools.partial(
        pl.kernel,
        out_type=out_sds,
        mesh=mesh,
        scratch_types=tuple(scratch),
        compiler_params=pltpu.CompilerParams(use_tc_tiling_on_sc=False),
    )
    def kern(src_hbm, dst_hbm, eb_hbm, table_hbm, out_hbm,
             acc, zbuf, ebs, sbig, dbig, *bufs):
        si = bufs[0:nbuf]
        di = bufs[nbuf:2 * nbuf]
        rb = bufs[2 * nbuf:3 * nbuf]
        gsem = bufs[3 * nbuf:4 * nbuf]
        ssem = bufs[4 * nbuf:5 * nbuf]

        c = lax.axis_index("c")
        s = lax.axis_index("s")
        wid = c * _NS + s

        pltpu.sync_copy(eb_hbm, ebs)

        def zfill(r, _):
            zbuf[r, pl.ds(0, 16)] = jnp.zeros((16,), jnp.float32)
            zbuf[r, pl.ds(16, 16)] = jnp.zeros((16,), jnp.float32)
            return 0
        lax.fori_loop(0, zrows, zfill, 0)
        z0 = s * acc_t
        done = 0
        while done < acc_t:
            step = min(zrows, acc_t - done)
            pltpu.sync_copy(zbuf.at[pl.ds(0, step)],
                            acc.at[pl.ds(z0 + done, step)])
            done += step
        plsc.subcore_barrier()

        ebv = ebs[pl.ds(wid, 16)]
        lo = ebv[0]
        hi = ebv[1]
        a0 = (lo // 8) * 8
        ngroups = ((hi - a0 + _CHUNK - 1) // _CHUNK + (nbuf - 1)) // nbuf

        for b in range(nbuf):
            for i in range(_CHUNK // 16):
                di[b][pl.ds(16 * i, 16)] = jnp.full((16,), dummy, jnp.int32)
                si[b][pl.ds(16 * i, 16)] = jnp.zeros((16,), jnp.int32)
            pltpu.async_copy(rb[b], acc.at[di[b]], ssem[b], add=True)

        def group(g, _):
            @pl.when(g % 2 == 0)
            def _():
                offg = pl.multiple_of(a0 + g * (nbuf * _CHUNK), 8)
                pltpu.sync_copy(src_hbm.at[pl.ds(offg, 8 * _CHUNK)], sbig)
                pltpu.sync_copy(dst_hbm.at[pl.ds(offg, 8 * _CHUNK)], dbig)
            loc0 = (g % 2) * (nbuf * _CHUNK)
            gd = []
            for b in range(nbuf):
                pltpu.make_async_copy(rb[b], acc.at[di[b]], ssem[b]).wait()
                off = a0 + (g * nbuf + b) * _CHUNK
                loc = loc0 + b * _CHUNK
                for i in range(_CHUNK // 16):
                    pos = lax.iota(jnp.int32, 16) + (off + 16 * i)
                    ok = (pos >= lo) & (pos < hi)
                    di[b][pl.ds(16 * i, 16)] = jnp.where(
                        ok, dbig[pl.ds(loc + 16 * i, 16)], dummy)
                    si[b][pl.ds(16 * i, 16)] = jnp.where(
                        ok, sbig[pl.ds(loc + 16 * i, 16)], 0)
                gd.append(pltpu.async_copy(
                    table_hbm.at[si[b]], rb[b], gsem[b]))
            for b in range(nbuf):
                gd[b].wait()
                pltpu.async_copy(rb[b], acc.at[di[b]], ssem[b], add=True)
            return 0

        lax.fori_loop(0, ngroups, group, 0)
        for b in range(nbuf):
            pltpu.make_async_copy(rb[b], acc.at[di[b]], ssem[b]).wait()
        plsc.subcore_barrier()

        start = pl.multiple_of(wid * rpt, 8)
        pltpu.sync_copy(acc.at[pl.ds(start, rpt)],
                        out_hbm.at[pl.ds(start, rpt)])

    return kern


_E_LEN = 801536
_ACC_N = 51200
_POOL_E_LEN = 51536
_POOL_ACC = 1024


def _edge_scatter(src_s, dst_s, eb, table):
    return _make_sc_scatter(_E_LEN, _ACC_N)(src_s, dst_s, eb, table)


def _pool_scatter(src_s, dst_s, eb, table):
    return _make_sc_scatter(_POOL_E_LEN, _POOL_ACC)(src_s, dst_s, eb, table)


def _csr_prep(src, dst, e_len, rpt):
    dst_s, src_s = lax.sort((dst, src), num_keys=1, is_stable=True)
    eb = jnp.searchsorted(
        dst_s, jnp.arange(33, dtype=jnp.int32) * rpt).astype(jnp.int32)
    eb = jnp.concatenate([eb, jnp.zeros((15,), jnp.int32)])
    pad = e_len - src.shape[0]
    src_s = jnp.concatenate([src_s, jnp.zeros((pad,), jnp.int32)])
    dst_s = jnp.concatenate([dst_s, jnp.zeros((pad,), jnp.int32)])
    return src_s, dst_s, eb


_RB = 5000


def _conv(x, aggs, feat, w1, b1, w2, b2):
    n_slabs = len(aggs)

    def f(*refs):
        x_ref = refs[0]
        a_refs = refs[1:1 + n_slabs]
        b1_ref, w1_ref, b2_ref, w2_ref = refs[1 + n_slabs:5 + n_slabs]
        u_ref, s1_ref, s2_ref = refs[5 + n_slabs:]
        slabs = [a[...] for a in a_refs]
        agg = slabs[0] if n_slabs == 1 else jnp.concatenate(slabs, axis=1)
        t = x_ref[...] + agg[:, :feat]
        y1 = jnp.maximum(jnp.dot(t, w1_ref[...],
                                 preferred_element_type=jnp.float32)
                         + b1_ref[...], 0.0)
        h = jnp.dot(y1, w2_ref[...], preferred_element_type=jnp.float32) \
            + b2_ref[...]
        u = jnp.maximum(h, 0.0)
        u_ref[...] = u

        @pl.when(pl.program_id(0) == 0)
        def _():
            s1_ref[...] = jnp.zeros_like(s1_ref)
            s2_ref[...] = jnp.zeros_like(s2_ref)
        s1_ref[...] += jnp.sum(u, axis=0, keepdims=True)
        s2_ref[...] += jnp.sum(u * u, axis=0, keepdims=True)

    node_in = pl.BlockSpec((_RB, feat), lambda i: (i, 0))
    node = pl.BlockSpec((_RB, _DIM), lambda i: (i, 0))
    vec = pl.BlockSpec((1, _DIM), lambda i: (0, 0))
    mat1 = pl.BlockSpec((feat, _DIM), lambda i: (0, 0))
    mat2 = pl.BlockSpec((_DIM, _DIM), lambda i: (0, 0))
    return pl.pallas_call(
        f,
        grid=(_N // _RB,),
        in_specs=[node_in] + [node] * n_slabs + [vec, mat1, vec, mat2],
        out_specs=(node, vec, vec),
        out_shape=(jax.ShapeDtypeStruct((_N, _DIM), jnp.float32),
                   jax.ShapeDtypeStruct((1, _DIM), jnp.float32),
                   jax.ShapeDtypeStruct((1, _DIM), jnp.float32)),
    )(x, *aggs, b1.reshape(1, _DIM), w1, b2.reshape(1, _DIM), w2)


def _bn_affine(u, s1, s2, gamma, beta):
    def f(u_ref, s1_ref, s2_ref, g_ref, be_ref, y_ref):
        mu = s1_ref[...] * (1.0 / _N)
        var = s2_ref[...] * (1.0 / _N) - mu * mu
        y_ref[...] = g_ref[...] * (u_ref[...] - mu) \
            / jnp.sqrt(var + _EPS) + be_ref[...]

    node = pl.BlockSpec((_RB, _DIM), lambda i: (i, 0))
    vec = pl.BlockSpec((1, _DIM), lambda i: (0, 0))
    return pl.pallas_call(
        f,
        grid=(_N // _RB,),
        in_specs=[node, vec, vec, vec, vec],
        out_specs=node,
        out_shape=jax.ShapeDtypeStruct((_N, _DIM), jnp.float32),
    )(u, s1, s2, gamma.reshape(1, _DIM), beta.reshape(1, _DIM))


def _head(p1, p2, target, params):
    d1, d2 = params["D1"], params["D2"]
    ins = [p1, p2, target,
           d1["fc"]["W"], d1["fc"]["b"].reshape(1, -1),
           d2["fc"]["W"], d2["fc"]["b"].reshape(1, -1),
           params["fc1_xt"]["W"], params["fc1_xt"]["b"].reshape(1, -1),
           params["fc1"]["W"], params["fc1"]["b"].reshape(1, -1),
           params["fc2"]["W"], params["fc2"]["b"].reshape(1, -1),
           params["out"]["W"], params["out"]["b"].reshape(1, -1)]

    def f(p1_ref, p2_ref, tg_ref,
          wf1, bf1, wf2, bf2, wxt, bxt, w1, b1_, w2, b2_, wo, bo, out_ref):
        dot = functools.partial(jnp.dot, preferred_element_type=jnp.float32)
        h1 = jnp.maximum(dot(p1_ref[...], wf1[...]) + bf1[...], 0.0)
        h2 = jnp.maximum(dot(p2_ref[...], wf2[...]) + bf2[...], 0.0)
        xt = dot(tg_ref[...], wxt[...]) + bxt[...]
        xc = jnp.concatenate([h1, h2, xt], axis=1)
        z = jnp.maximum(dot(xc, w1[...]) + b1_[...], 0.0)
        z = jnp.maximum(dot(z, w2[...]) + b2_[...], 0.0)
        out_ref[...] = dot(z, wo[...]) + bo[...]

    return pl.pallas_call(
        f,
        out_shape=jax.ShapeDtypeStruct((_B, 1), jnp.float32),
    )(*ins)


def _branch(x, edge_index, batch, bp):
    src_s, dst_s, eb = _csr_prep(edge_index[0], edge_index[1],
                                 _E_LEN, _ACC_N // _NW)
    layers = bp["layers"]

    xp = jnp.pad(x, ((0, 0), (0, 96 - _FEAT)))
    aggs = [_edge_scatter(src_s, dst_s, eb, xp[:, 32 * k:32 * (k + 1)])
            for k in range(3)]
    lp = layers[0]
    u, s1, s2 = _conv(x, aggs, _FEAT, lp["lin1"]["W"], lp["lin1"]["b"],
                      lp["lin2"]["W"], lp["lin2"]["b"])
    for li in range(1, 5):
        lp = layers[li]
        prev = layers[li - 1]
        xl = _bn_affine(u, s1, s2, prev["gamma"], prev["beta"])
        agg = _edge_scatter(src_s, dst_s, eb, xl)
        u, s1, s2 = _conv(xl, [agg], _DIM, lp["lin1"]["W"], lp["lin1"]["b"],
                          lp["lin2"]["W"], lp["lin2"]["b"])
    x5 = _bn_affine(u, s1, s2, layers[4]["gamma"], layers[4]["beta"])

    psrc = jnp.arange(_N, dtype=jnp.int32)
    peb = jnp.searchsorted(
        batch, jnp.arange(33, dtype=jnp.int32) * (_POOL_ACC // _NW)
    ).astype(jnp.int32)
    peb = jnp.concatenate([peb, jnp.zeros((15,), jnp.int32)])
    pad = _POOL_E_LEN - _N
    psrc = jnp.concatenate([psrc, jnp.zeros((pad,), jnp.int32)])
    pdst = jnp.concatenate([batch, jnp.zeros((pad,), jnp.int32)])
    pooled = _pool_scatter(psrc, pdst, peb, x5)
    return pooled[:_B]


def kernel(x1, edge_index_1, batch1, x2, edge_index_2, batch2, target, params):
    p1 = _branch(x1, edge_index_1, batch1, params["D1"])
    p2 = _branch(x2, edge_index_2, batch2, params["D2"])
    return _head(p1, p2, target, params)
